# Initial kernel scaffold; baseline (speedup 1.0000x reference)
import functools

import jax
import jax.numpy as jnp
from jax import lax
from jax.experimental import pallas as pl
from jax.experimental.pallas import tpu as pltpu
from jax.experimental.pallas import tpu_sc as plsc

B = 800000
D = 36
NC = 2
NS = 16
NW = NC * NS
BPW = B // NW
C = 1000
NCHUNK = BPW // C


def _body(idx_hbm, tbl_hbm, out_hbm, idx_v, rows_v, sem):
    wid = lax.axis_index("s") * NC + lax.axis_index("c")
    base = wid * BPW

    def chunk(i, carry):
        off = base + i * C
        pltpu.sync_copy(idx_hbm.at[pl.ds(off, C)], idx_v)
        pltpu.async_copy(tbl_hbm.at[idx_v], rows_v, sem).wait()
        pltpu.sync_copy(rows_v, out_hbm.at[pl.ds(off, C)])
        return carry

    lax.fori_loop(0, NCHUNK, chunk, 0)


_mesh = plsc.VectorSubcoreMesh(core_axis_name="c", subcore_axis_name="s")

_gather = functools.partial(
    pl.kernel,
    mesh=_mesh,
    out_type=jax.ShapeDtypeStruct((B, D), jnp.float32),
    scratch_types=[
        pltpu.VMEM((C,), jnp.int32),
        pltpu.VMEM((C, D), jnp.float32),
        pltpu.SemaphoreType.DMA,
    ],
)(_body)


def kernel(atomic_numbers, gate_weight):
    return _gather(atomic_numbers, gate_weight)


# speedup vs baseline: 1.8088x; 1.8088x over previous
import functools

import jax
import jax.numpy as jnp
from jax import lax
from jax.experimental import pallas as pl
from jax.experimental.pallas import tpu as pltpu
from jax.experimental.pallas import tpu_sc as plsc

B = 800000
D = 36
DP = 40
NC = 2
NS = 16
NW = NC * NS
G = 40
NG = B // G
GPW = NG // NW
INNER = 25
OUTER = GPW // INNER


def _body(idx_hbm, tbl_hbm, out_hbm, idx_v, rows_v, sem):
    wid = lax.axis_index("s") * NC + lax.axis_index("c")
    g0 = wid * GPW
    pltpu.sync_copy(idx_hbm.at[pl.ds(g0, GPW)], idx_v)

    def outer(j, c):
        def inner(k, c2):
            s = j * INNER + k
            pltpu.async_copy(tbl_hbm.at[idx_v.at[s]], rows_v, sem).wait()
            pltpu.sync_copy(rows_v, out_hbm.at[g0 + s])
            return c2

        lax.fori_loop(0, INNER, inner, 0)
        return c

    lax.fori_loop(0, OUTER, outer, 0)


_mesh = plsc.VectorSubcoreMesh(core_axis_name="c", subcore_axis_name="s")

_gather = functools.partial(
    pl.kernel,
    mesh=_mesh,
    out_type=jax.ShapeDtypeStruct((NG, G, DP), jnp.float32),
    scratch_types=[
        pltpu.VMEM((GPW, G), jnp.int32),
        pltpu.VMEM((G, DP), jnp.float32),
        pltpu.SemaphoreType.DMA,
    ],
    compiler_params=pltpu.CompilerParams(use_tc_tiling_on_sc=False),
)(_body)


def kernel(atomic_numbers, gate_weight):
    tbl = jnp.pad(gate_weight, ((0, 0), (0, DP - D)))
    out = _gather(atomic_numbers.reshape(NG, G), tbl)
    return out[:, :, :D].reshape(B, D)
